# Initial kernel scaffold; baseline (speedup 1.0000x reference)
import jax
import jax.numpy as jnp
from jax.experimental import pallas as pl


def kernel(x, embeddings):
    raise NotImplementedError("write your pallas kernel here")


# speedup vs baseline: 1.5460x; 1.5460x over previous
import functools

import jax
import jax.numpy as jnp
from jax import lax
from jax.experimental import pallas as pl
from jax.experimental.pallas import tpu as pltpu
from jax.experimental.pallas import tpu_sc as plsc

EMBEDDING_DIM = 32
CHUNK = 1024


@functools.cache
def _build(B, D):
    info = plsc.get_sparse_core_info()
    NC, NS = info.num_cores, info.num_subcores
    NW = NC * NS
    assert B % NW == 0
    b_per_w = B // NW
    assert b_per_w % CHUNK == 0
    n_chunks = b_per_w // CHUNK

    mesh = plsc.VectorSubcoreMesh(core_axis_name="c", subcore_axis_name="s")

    @functools.partial(
        pl.kernel,
        mesh=mesh,
        out_type=jax.ShapeDtypeStruct((B, D), jnp.float32),
        scratch_types=[
            pltpu.VMEM((CHUNK,), jnp.int32),
            pltpu.VMEM((CHUNK, D), jnp.float32),
            pltpu.SemaphoreType.DMA,
        ],
        compiler_params=pltpu.CompilerParams(use_tc_tiling_on_sc=False),
    )
    def gather_kernel(idx_hbm, table_hbm, out_hbm, idx_v, rows_v, sem):
        wid = lax.axis_index("s") * NC + lax.axis_index("c")
        base = wid * b_per_w

        def body(i, carry):
            off = base + i * CHUNK
            pltpu.sync_copy(idx_hbm.at[pl.ds(off, CHUNK)], idx_v)
            pltpu.async_copy(table_hbm.at[idx_v], rows_v, sem).wait()
            pltpu.sync_copy(rows_v, out_hbm.at[pl.ds(off, CHUNK)])
            return carry

        lax.fori_loop(0, n_chunks, body, 0)

    return gather_kernel


def kernel(x, embeddings):
    n, s = x.shape
    B = n * s
    xf = x.reshape(B).astype(jnp.int32)
    out = _build(B, EMBEDDING_DIM)(xf, embeddings)
    return out.reshape(n, s, EMBEDDING_DIM)


# speedup vs baseline: 1.5678x; 1.0141x over previous
import functools

import jax
import jax.numpy as jnp
from jax import lax
from jax.experimental import pallas as pl
from jax.experimental.pallas import tpu as pltpu
from jax.experimental.pallas import tpu_sc as plsc

EMBEDDING_DIM = 32
CHUNK = 1664


@functools.cache
def _build(B, D):
    info = plsc.get_sparse_core_info()
    NC, NS = info.num_cores, info.num_subcores
    NW = NC * NS
    assert B % NW == 0
    b_per_w = B // NW
    assert b_per_w % CHUNK == 0
    n_chunks = b_per_w // CHUNK

    mesh = plsc.VectorSubcoreMesh(core_axis_name="c", subcore_axis_name="s")

    @functools.partial(
        pl.kernel,
        mesh=mesh,
        out_type=jax.ShapeDtypeStruct((B, D), jnp.float32),
        scratch_types=[
            pltpu.VMEM((CHUNK,), jnp.int32),
            pltpu.VMEM((CHUNK,), jnp.int32),
            pltpu.VMEM((CHUNK, D), jnp.float32),
            pltpu.VMEM((CHUNK, D), jnp.float32),
            pltpu.SemaphoreType.DMA,
            pltpu.SemaphoreType.DMA,
            pltpu.SemaphoreType.DMA,
            pltpu.SemaphoreType.DMA,
            pltpu.SemaphoreType.DMA,
            pltpu.SemaphoreType.DMA,
        ],
        compiler_params=pltpu.CompilerParams(use_tc_tiling_on_sc=False),
    )
    def gather_kernel(idx_hbm, table_hbm, out_hbm,
                      i0, i1, r0, r1, si0, si1, sg0, sg1, ss0, ss1):
        idx_bufs, rows_bufs = [i0, i1], [r0, r1]
        sem_i, sem_g, sem_s = [si0, si1], [sg0, sg1], [ss0, ss1]
        wid = lax.axis_index("s") * NC + lax.axis_index("c")
        base = wid * b_per_w

        def idx_start(i):
            off = base + i * CHUNK
            return pltpu.async_copy(
                idx_hbm.at[pl.ds(off, CHUNK)], idx_bufs[i % 2], sem_i[i % 2])

        def gather_start(i):
            return pltpu.async_copy(
                table_hbm.at[idx_bufs[i % 2]], rows_bufs[i % 2], sem_g[i % 2])

        def store_start(i):
            off = base + i * CHUNK
            return pltpu.async_copy(
                rows_bufs[i % 2], out_hbm.at[pl.ds(off, CHUNK)], sem_s[i % 2])

        d_idx, d_g, d_s = {}, {}, {}
        d_idx[0] = idx_start(0)
        if n_chunks > 1:
            d_idx[1] = idx_start(1)
        d_idx[0].wait()
        d_g[0] = gather_start(0)
        for i in range(n_chunks):
            d_g[i].wait()
            d_s[i] = store_start(i)
            if i + 1 < n_chunks:
                if i - 1 >= 0:
                    d_s[i - 1].wait()
                d_idx[i + 1].wait()
                d_g[i + 1] = gather_start(i + 1)
            if i + 2 < n_chunks:
                d_idx[i + 2] = idx_start(i + 2)
        if n_chunks >= 2:
            d_s[n_chunks - 2].wait()
        d_s[n_chunks - 1].wait()

    return gather_kernel


def kernel(x, embeddings):
    n, s = x.shape
    B = n * s
    xf = x.reshape(B).astype(jnp.int32)
    out = _build(B, EMBEDDING_DIM)(xf, embeddings)
    return out.reshape(n, s, EMBEDDING_DIM)
